# baseline (speedup 1.0000x reference)
import functools

import jax
import jax.numpy as jnp
from jax import lax
from jax.experimental import pallas as pl
from jax.experimental.pallas import tpu as pltpu
from jax.experimental.pallas import tpu_sc as plsc

D = 64
GROUP = 128
CG = 4
NB = 3
NC, NS = 2, 16
NW = NC * NS


@functools.lru_cache(maxsize=None)
def _build(B):
    assert B % (GROUP * NW) == 0
    n_groups = B // GROUP
    g_per_w = n_groups // NW
    assert g_per_w % CG == 0
    n_chunks = g_per_w // CG
    rows_per_chunk = CG * GROUP

    mesh = plsc.VectorSubcoreMesh(core_axis_name="c", subcore_axis_name="s")

    @functools.partial(
        pl.kernel,
        out_type=jax.ShapeDtypeStruct((B, D), jnp.float32),
        mesh=mesh,
        compiler_params=pltpu.CompilerParams(use_tc_tiling_on_sc=False),
        scratch_types=[
            pltpu.VMEM((g_per_w, GROUP), jnp.int32),
            pltpu.VMEM((NB, rows_per_chunk, D), jnp.float32),
        ]
        + [pltpu.SemaphoreType.DMA] * (2 * NB),
    )
    def emb_kernel(idx_hbm, table_hbm, out_hbm, idx_v, rows_v, *sems):
        gsems = sems[:NB]
        wsems = sems[NB:]
        wid = lax.axis_index("s") * NC + lax.axis_index("c")
        g0 = wid * g_per_w

        pltpu.sync_copy(idx_hbm.at[pl.ds(g0, g_per_w)], idx_v)

        def g_descs(ch, b):
            return [
                pltpu.make_async_copy(
                    table_hbm.at[idx_v.at[ch * CG + q]],
                    rows_v.at[b, pl.ds(q * GROUP, GROUP)],
                    gsems[b],
                )
                for q in range(CG)
            ]

        def g_start(ch, b):
            for c in g_descs(ch, b):
                c.start()

        def g_wait(ch, b):
            for c in g_descs(ch, b):
                c.wait()

        def w_desc(ch, b):
            return pltpu.make_async_copy(
                rows_v.at[b],
                out_hbm.at[pl.ds((g0 + ch * CG) * GROUP, rows_per_chunk)],
                wsems[b],
            )

        g_start(0, 0)
        g_start(1, 1)
        g_wait(0, 0)
        w_desc(0, 0).start()
        g_start(2, 2)

        def body(m, _):
            j0 = 3 * m + 1
            for t, (b, bp) in enumerate(((1, 0), (2, 1), (0, 2))):
                j = j0 + t
                g_wait(j, b)
                w_desc(j, b).start()
                w_desc(j - 1, bp).wait()
                g_start(j + 2, bp)
            return 0

        lax.fori_loop(0, (n_chunks - 5) // 3, body, 0)

        j = n_chunks - 4
        g_wait(j, 1)
        w_desc(j, 1).start()
        w_desc(j - 1, 0).wait()
        g_start(j + 2, 0)
        j += 1
        g_wait(j, 2)
        w_desc(j, 2).start()
        w_desc(j - 1, 1).wait()
        g_start(j + 2, 1)
        j += 1
        g_wait(j, 0)
        w_desc(j, 0).start()
        j += 1
        g_wait(j, 1)
        w_desc(j, 1).start()
        w_desc(n_chunks - 3, 2).wait()
        w_desc(n_chunks - 2, 0).wait()
        w_desc(n_chunks - 1, 1).wait()

    return emb_kernel


def kernel(token_ids, emb):
    s0, s1 = token_ids.shape
    B = s0 * s1
    idx = token_ids.reshape(B // GROUP, GROUP).astype(jnp.int32)
    out = _build(B)(idx, emb)
    return out.reshape(s0, s1, D)
